# baseline (speedup 1.0000x reference)
import functools

import jax
import jax.numpy as jnp
from jax import lax
from jax.experimental import pallas as pl
from jax.experimental.pallas import tpu as pltpu
from jax.experimental.pallas import tpu_sc as plsc

B, S, H, FF, E = 1, 2048, 1024, 512, 16
TOP_K, N_GROUP, TOPK_GROUP = 2, 8, 4
SCALING = 2.5
T = B * S

BT = 256
NBLK_R = T * TOP_K // BT
NBLK = NBLK_R + E
NPOS = NBLK * BT
NBLK_S = T // BT
NBLK_ALL = NBLK + NBLK_S


def _router_kernel(x_ref, wr_ref, metaf_ref, metai_ref, blk_ref):
    x = x_ref[...]
    wr = wr_ref[...]
    logits = lax.dot_general(x, wr, (((1,), (1,)), ((), ())),
                             preferred_element_type=jnp.float32)
    scores = jax.nn.sigmoid(logits)

    iota = lax.broadcasted_iota(jnp.int32, (T, E), 1)
    r16 = lax.broadcasted_iota(jnp.int32, (E, E), 0)
    c16 = lax.broadcasted_iota(jnp.int32, (E, E), 1)
    partner_idx = r16 + 1 - 2 * (r16 % 2)
    P = (partner_idx == c16).astype(jnp.float32)
    partner = lax.dot_general(scores, P, (((1,), (0,)), ((), ())),
                              preferred_element_type=jnp.float32)
    gs = jnp.maximum(scores, partner)
    giota = iota // 2

    group_mask = jnp.zeros((T, E), dtype=jnp.bool_)
    g = gs
    for _ in range(TOPK_GROUP):
        m = jnp.max(g, axis=-1, keepdims=True)
        cand = jnp.where(g == m, giota, N_GROUP)
        sel = jnp.min(cand, axis=-1, keepdims=True)
        hit = giota == sel
        group_mask = group_mask | hit
        g = jnp.where(hit, -1.0, g)

    routed = jnp.where(group_mask, scores, 0.0)

    r = routed
    wsum = jnp.zeros((T, 1), dtype=jnp.float32)
    picks = []
    for _ in range(TOP_K):
        m = jnp.max(r, axis=-1, keepdims=True)
        cand = jnp.where(r == m, iota, E)
        sel = jnp.min(cand, axis=-1, keepdims=True)
        hit = iota == sel
        picks.append((hit, m))
        wsum = wsum + m
        r = jnp.where(hit, -1.0, r)

    scale = SCALING / jnp.maximum(wsum, 1e-9)
    (hit0, m0), (hit1, m1) = picks
    metaf_ref[...] = jnp.concatenate([m0 * scale, m1 * scale], axis=1)

    sel01 = hit0.astype(jnp.float32) + hit1.astype(jnp.float32)
    rT = lax.broadcasted_iota(jnp.int32, (T, T), 0)
    cT = lax.broadcasted_iota(jnp.int32, (T, T), 1)
    Lt = (cT < rT).astype(jnp.float32)
    rank = lax.dot_general(Lt, sel01, (((1,), (0,)), ((), ())),
                           preferred_element_type=jnp.float32)
    count = jnp.sum(sel01, axis=0, keepdims=True)
    padded = jnp.ceil(count * (1.0 / BT)) * BT
    U = (r16 < c16).astype(jnp.float32)
    off = lax.dot_general(padded, U, (((1,), (0,)), ((), ())),
                          preferred_element_type=jnp.float32)
    posmat = off + rank
    pos0 = jnp.sum(jnp.where(hit0, posmat, 0.0), axis=-1, keepdims=True)
    pos1 = jnp.sum(jnp.where(hit1, posmat, 0.0), axis=-1, keepdims=True)
    metai_ref[...] = jnp.concatenate([pos0, pos1], axis=1).astype(jnp.int32)

    total = jnp.sum(padded, axis=-1, keepdims=True)
    ib = lax.broadcasted_iota(jnp.int32, (NBLK_ALL, 1), 0)
    bidx = (ib * BT).astype(jnp.float32)
    offb = jnp.broadcast_to(off, (NBLK_ALL, E))
    be_r = jnp.sum((bidx >= offb).astype(jnp.int32), axis=-1,
                   keepdims=True) - 1
    is_shared = ib >= NBLK
    be = jnp.where(is_shared, E - 1, jnp.minimum(be_r, E - 1))
    valid = (is_shared | (bidx < total)).astype(jnp.int32)
    blk_ref[...] = jnp.concatenate([be, valid], axis=1)


def _dispatch_kernel(tok_hbm, pos0_hbm, pos1_hbm, xs_hbm,
                     tv, i0, i1, sem0, sem1):
    NC = 2
    wid = lax.axis_index("s") * NC + lax.axis_index("c")
    base = wid * (T // 32)
    CH = 64
    pltpu.sync_copy(tok_hbm.at[pl.ds(base, CH)], tv)
    pltpu.sync_copy(pos0_hbm.at[pl.ds(base, CH)], i0)
    pltpu.sync_copy(pos1_hbm.at[pl.ds(base, CH)], i1)
    c0 = pltpu.async_copy(tv, xs_hbm.at[i0], sem0)
    c1 = pltpu.async_copy(tv, xs_hbm.at[i1], sem1)
    c0.wait()
    c1.wait()


def _ffn_kernel(be_ref, valid_ref, xs_ref, tok_ref, g_ref, u_ref, d_ref,
                sg_ref, su_ref, sd_ref, out_ref):
    b = pl.program_id(0)

    def gated(x, gw, uw, dw):
        a = lax.dot_general(x, gw, (((1,), (1,)), ((), ())),
                            preferred_element_type=jnp.float32)
        bb = lax.dot_general(x, uw, (((1,), (1,)), ((), ())),
                             preferred_element_type=jnp.float32)
        h = a * jax.nn.sigmoid(a) * bb
        return lax.dot_general(h, dw, (((1,), (1,)), ((), ())),
                               preferred_element_type=jnp.float32)

    @pl.when(jnp.logical_and(b < NBLK, valid_ref[b] == 1))
    def _():
        out_ref[...] = gated(xs_ref[...], g_ref[0], u_ref[0], d_ref[0])

    @pl.when(b >= NBLK)
    def _():
        out_ref[...] = gated(tok_ref[...], sg_ref[...], su_ref[...],
                             sd_ref[...])


def _combine_kernel(exp_hbm, pos0_hbm, pos1_hbm, w0_hbm, w1_hbm,
                    out_hbm, r0, r1, sh, ov, i0, i1, wv0, wv1, sem0, sem1):
    NC = 2
    wid = lax.axis_index("s") * NC + lax.axis_index("c")
    tbase = wid * (T // 32)
    CH = 16
    NITER = (T // 32) // CH

    pltpu.sync_copy(pos0_hbm.at[pl.ds(tbase, T // 32)], i0)
    pltpu.sync_copy(pos1_hbm.at[pl.ds(tbase, T // 32)], i1)
    pltpu.sync_copy(w0_hbm.at[pl.ds(tbase, T // 32)], wv0.at[pl.ds(0, T // 32)])
    pltpu.sync_copy(w1_hbm.at[pl.ds(tbase, T // 32)], wv1.at[pl.ds(0, T // 32)])

    def chunk(it, carry):
        base = tbase + it * CH
        c0 = pltpu.async_copy(exp_hbm.at[i0.at[pl.ds(it * CH, CH)]], r0, sem0)
        c1 = pltpu.async_copy(exp_hbm.at[i1.at[pl.ds(it * CH, CH)]], r1, sem1)
        pltpu.sync_copy(exp_hbm.at[pl.ds(NPOS + base, CH)], sh)
        c0.wait()
        c1.wait()

        def per_token(i, carry2):
            w0s = jnp.full((16,), wv0[pl.ds(it * CH + i, 16)][0],
                           dtype=jnp.float32)
            w1s = jnp.full((16,), wv1[pl.ds(it * CH + i, 16)][0],
                           dtype=jnp.float32)
            for c in range(H // 16):
                s = pl.ds(c * 16, 16)
                ov[i, s] = r0[i, s] * w0s + r1[i, s] * w1s + sh[i, s]
            return carry2

        lax.fori_loop(0, CH, per_token, 0)
        pltpu.sync_copy(ov, out_hbm.at[pl.ds(base, CH)])
        return carry

    lax.fori_loop(0, NITER, chunk, 0)


def kernel(x, W_router, gate_w, up_w, down_w, shared_gate, shared_up, shared_down):
    b, s, h = x.shape
    tokens = x.reshape(T, H)

    metaf, metai, blk = pl.pallas_call(
        _router_kernel,
        out_shape=(
            jax.ShapeDtypeStruct((T, 2), jnp.float32),
            jax.ShapeDtypeStruct((T, 2), jnp.int32),
            jax.ShapeDtypeStruct((NBLK_ALL, 2), jnp.int32),
        ),
    )(tokens, W_router)

    w0, w1 = metaf[:, 0], metaf[:, 1]
    pos0, pos1 = metai[:, 0], metai[:, 1]
    be, valid = blk[:, 0], blk[:, 1]

    mesh = plsc.VectorSubcoreMesh(core_axis_name="c", subcore_axis_name="s")

    xs = pl.kernel(
        _dispatch_kernel,
        mesh=mesh,
        out_type=jax.ShapeDtypeStruct((NPOS, H), jnp.float32),
        scratch_types=[
            pltpu.VMEM((64, H), jnp.float32),
            pltpu.VMEM((64,), jnp.int32),
            pltpu.VMEM((64,), jnp.int32),
            pltpu.SemaphoreType.DMA,
            pltpu.SemaphoreType.DMA,
        ],
    )(tokens, pos0, pos1)

    exp_out = pl.pallas_call(
        _ffn_kernel,
        grid_spec=pltpu.PrefetchScalarGridSpec(
            num_scalar_prefetch=2,
            grid=(NBLK_ALL,),
            in_specs=[
                pl.BlockSpec((BT, H),
                             lambda bb, be_, v_: (jnp.minimum(bb, NBLK - 1), 0)),
                pl.BlockSpec((BT, H),
                             lambda bb, be_, v_: (jnp.maximum(bb - NBLK, 0), 0)),
                pl.BlockSpec((1, FF, H), lambda bb, be_, v_: (be_[bb], 0, 0)),
                pl.BlockSpec((1, FF, H), lambda bb, be_, v_: (be_[bb], 0, 0)),
                pl.BlockSpec((1, H, FF), lambda bb, be_, v_: (be_[bb], 0, 0)),
                pl.BlockSpec((FF, H), lambda bb, be_, v_: (0, 0)),
                pl.BlockSpec((FF, H), lambda bb, be_, v_: (0, 0)),
                pl.BlockSpec((H, FF), lambda bb, be_, v_: (0, 0)),
            ],
            out_specs=pl.BlockSpec((BT, H), lambda bb, be_, v_: (bb, 0)),
        ),
        out_shape=jax.ShapeDtypeStruct((NPOS + T, H), jnp.float32),
    )(be, valid, xs, tokens, gate_w, up_w, down_w,
      shared_gate, shared_up, shared_down)

    out = pl.kernel(
        _combine_kernel,
        mesh=mesh,
        out_type=jax.ShapeDtypeStruct((T, H), jnp.float32),
        scratch_types=[
            pltpu.VMEM((16, H), jnp.float32),
            pltpu.VMEM((16, H), jnp.float32),
            pltpu.VMEM((16, H), jnp.float32),
            pltpu.VMEM((16, H), jnp.float32),
            pltpu.VMEM((64,), jnp.int32),
            pltpu.VMEM((64,), jnp.int32),
            pltpu.VMEM((80,), jnp.float32),
            pltpu.VMEM((80,), jnp.float32),
            pltpu.SemaphoreType.DMA,
            pltpu.SemaphoreType.DMA,
        ],
    )(exp_out, pos0, pos1, w0, w1)

    return out.reshape(b, s, h)


# speedup vs baseline: 1.0999x; 1.0999x over previous
import functools

import jax
import jax.numpy as jnp
from jax import lax
from jax.experimental import pallas as pl
from jax.experimental.pallas import tpu as pltpu
from jax.experimental.pallas import tpu_sc as plsc

B, S, H, FF, E = 1, 2048, 1024, 512, 16
TOP_K, N_GROUP, TOPK_GROUP = 2, 8, 4
SCALING = 2.5
T = B * S

BT = 256
NBLK_R = T * TOP_K // BT
NBLK = NBLK_R + E
NPOS = NBLK * BT


def _router_kernel(x_ref, wr_ref, metaf_ref, metai_ref, blk_ref):
    x = x_ref[...]
    wr = wr_ref[...]
    logits = lax.dot_general(x, wr, (((1,), (1,)), ((), ())),
                             preferred_element_type=jnp.float32)
    scores = jax.nn.sigmoid(logits)

    iota = lax.broadcasted_iota(jnp.int32, (T, E), 1)
    r16 = lax.broadcasted_iota(jnp.int32, (E, E), 0)
    c16 = lax.broadcasted_iota(jnp.int32, (E, E), 1)
    partner_idx = r16 + 1 - 2 * (r16 % 2)
    P = (partner_idx == c16).astype(jnp.float32)
    partner = lax.dot_general(scores, P, (((1,), (0,)), ((), ())),
                              preferred_element_type=jnp.float32)
    gs = jnp.maximum(scores, partner)
    giota = iota // 2

    group_mask = jnp.zeros((T, E), dtype=jnp.bool_)
    g = gs
    for _ in range(TOPK_GROUP):
        m = jnp.max(g, axis=-1, keepdims=True)
        cand = jnp.where(g == m, giota, N_GROUP)
        sel = jnp.min(cand, axis=-1, keepdims=True)
        hit = giota == sel
        group_mask = group_mask | hit
        g = jnp.where(hit, -1.0, g)

    routed = jnp.where(group_mask, scores, 0.0)

    r = routed
    wsum = jnp.zeros((T, 1), dtype=jnp.float32)
    picks = []
    for _ in range(TOP_K):
        m = jnp.max(r, axis=-1, keepdims=True)
        cand = jnp.where(r == m, iota, E)
        sel = jnp.min(cand, axis=-1, keepdims=True)
        hit = iota == sel
        picks.append((hit, m))
        wsum = wsum + m
        r = jnp.where(hit, -1.0, r)

    scale = SCALING / jnp.maximum(wsum, 1e-9)
    (hit0, m0), (hit1, m1) = picks
    metaf_ref[...] = jnp.concatenate([m0 * scale, m1 * scale], axis=1)

    sel01 = hit0.astype(jnp.float32) + hit1.astype(jnp.float32)
    rT = lax.broadcasted_iota(jnp.int32, (T, T), 0)
    cT = lax.broadcasted_iota(jnp.int32, (T, T), 1)
    Lt = (cT < rT).astype(jnp.float32)
    rank = lax.dot_general(Lt, sel01, (((1,), (0,)), ((), ())),
                           preferred_element_type=jnp.float32)
    count = jnp.sum(sel01, axis=0, keepdims=True)
    padded = jnp.ceil(count * (1.0 / BT)) * BT
    U = (r16 < c16).astype(jnp.float32)
    off = lax.dot_general(padded, U, (((1,), (0,)), ((), ())),
                          preferred_element_type=jnp.float32)
    posmat = off + rank
    pos0 = jnp.sum(jnp.where(hit0, posmat, 0.0), axis=-1, keepdims=True)
    pos1 = jnp.sum(jnp.where(hit1, posmat, 0.0), axis=-1, keepdims=True)
    metai_ref[...] = jnp.concatenate([pos0, pos1], axis=1).astype(jnp.int32)

    total = jnp.sum(padded, axis=-1, keepdims=True)
    bidx = (lax.broadcasted_iota(jnp.int32, (NBLK, 1), 0)
            * BT).astype(jnp.float32)
    offb = jnp.broadcast_to(off, (NBLK, E))
    be = jnp.sum((bidx >= offb).astype(jnp.int32), axis=-1,
                 keepdims=True) - 1
    valid = (bidx < total).astype(jnp.int32)
    blk_ref[...] = jnp.concatenate([jnp.minimum(be, E - 1), valid], axis=1)


def _dispatch_kernel(tok_hbm, pos0_hbm, pos1_hbm, xs_hbm,
                     tv, i0, i1, sem0, sem1):
    NC = 2
    wid = lax.axis_index("s") * NC + lax.axis_index("c")
    base = wid * (T // 32)
    CH = 64
    pltpu.sync_copy(tok_hbm.at[pl.ds(base, CH)], tv)
    pltpu.sync_copy(pos0_hbm.at[pl.ds(base, CH)], i0)
    pltpu.sync_copy(pos1_hbm.at[pl.ds(base, CH)], i1)
    c0 = pltpu.async_copy(tv, xs_hbm.at[i0], sem0)
    c1 = pltpu.async_copy(tv, xs_hbm.at[i1], sem1)
    c0.wait()
    c1.wait()


def _shared_ffn_kernel(x_ref, g_ref, u_ref, d_ref, out_ref):
    x = x_ref[...]
    a = lax.dot_general(x, g_ref[...], (((1,), (1,)), ((), ())),
                        preferred_element_type=jnp.float32)
    bb = lax.dot_general(x, u_ref[...], (((1,), (1,)), ((), ())),
                         preferred_element_type=jnp.float32)
    h = a * jax.nn.sigmoid(a) * bb
    out_ref[...] = lax.dot_general(h, d_ref[...], (((1,), (1,)), ((), ())),
                                   preferred_element_type=jnp.float32)


def _ffn_kernel(be_ref, valid_ref, x_ref, g_ref, u_ref, d_ref, out_ref):
    b = pl.program_id(0)

    @pl.when(valid_ref[b] == 1)
    def _():
        x = x_ref[...]
        gw = g_ref[0]
        uw = u_ref[0]
        dw = d_ref[0]
        a = lax.dot_general(x, gw, (((1,), (1,)), ((), ())),
                            preferred_element_type=jnp.float32)
        bb = lax.dot_general(x, uw, (((1,), (1,)), ((), ())),
                             preferred_element_type=jnp.float32)
        h = a * jax.nn.sigmoid(a) * bb
        out_ref[...] = lax.dot_general(h, dw, (((1,), (1,)), ((), ())),
                                       preferred_element_type=jnp.float32)


CH4 = 16
NIT4 = (T // 32) // CH4


def _combine_kernel(exp_hbm, sh_hbm, pos0_hbm, pos1_hbm, w0_hbm, w1_hbm,
                    out_hbm, r0a, r0b, r1a, r1b, sha, shb, ov,
                    i0, i1, wv0, wv1, sem0, sem1):
    NC = 2
    wid = lax.axis_index("s") * NC + lax.axis_index("c")
    tbase = wid * (T // 32)

    pltpu.sync_copy(pos0_hbm.at[pl.ds(tbase, T // 32)], i0)
    pltpu.sync_copy(pos1_hbm.at[pl.ds(tbase, T // 32)], i1)
    pltpu.sync_copy(w0_hbm.at[pl.ds(tbase, T // 32)],
                    wv0.at[pl.ds(0, T // 32)])
    pltpu.sync_copy(w1_hbm.at[pl.ds(tbase, T // 32)],
                    wv1.at[pl.ds(0, T // 32)])

    bufs = [(r0a, r1a, sha), (r0b, r1b, shb)]

    def fetch(it, sem):
        r0, r1, sh = bufs[it % 2]
        base = tbase + it * CH4
        c0 = pltpu.async_copy(exp_hbm.at[i0.at[pl.ds(it * CH4, CH4)]],
                              r0, sem)
        c1 = pltpu.async_copy(exp_hbm.at[i1.at[pl.ds(it * CH4, CH4)]],
                              r1, sem)
        c2 = pltpu.async_copy(sh_hbm.at[pl.ds(base, CH4)], sh, sem)
        return (c0, c1, c2)

    pending = fetch(0, sem0)
    sems = [sem0, sem1]
    for it in range(NIT4):
        for c in pending:
            c.wait()
        if it + 1 < NIT4:
            pending = fetch(it + 1, sems[(it + 1) % 2])
        r0, r1, sh = bufs[it % 2]

        def per_token(i, carry2, _it=it, _r0=r0, _r1=r1, _sh=sh):
            w0s = jnp.full((16,), wv0[pl.ds(_it * CH4 + i, 16)][0],
                           dtype=jnp.float32)
            w1s = jnp.full((16,), wv1[pl.ds(_it * CH4 + i, 16)][0],
                           dtype=jnp.float32)
            for c in range(H // 16):
                s = pl.ds(c * 16, 16)
                ov[i, s] = _r0[i, s] * w0s + _r1[i, s] * w1s + _sh[i, s]
            return carry2

        lax.fori_loop(0, CH4, per_token, 0)
        pltpu.sync_copy(ov, out_hbm.at[pl.ds(tbase + it * CH4, CH4)])


def kernel(x, W_router, gate_w, up_w, down_w, shared_gate, shared_up, shared_down):
    b, s, h = x.shape
    tokens = x.reshape(T, H)

    metaf, metai, blk = pl.pallas_call(
        _router_kernel,
        out_shape=(
            jax.ShapeDtypeStruct((T, 2), jnp.float32),
            jax.ShapeDtypeStruct((T, 2), jnp.int32),
            jax.ShapeDtypeStruct((NBLK, 2), jnp.int32),
        ),
    )(tokens, W_router)

    w0, w1 = metaf[:, 0], metaf[:, 1]
    pos0, pos1 = metai[:, 0], metai[:, 1]
    be, valid = blk[:, 0], blk[:, 1]

    mesh = plsc.VectorSubcoreMesh(core_axis_name="c", subcore_axis_name="s")

    xs = pl.kernel(
        _dispatch_kernel,
        mesh=mesh,
        out_type=jax.ShapeDtypeStruct((NPOS, H), jnp.float32),
        scratch_types=[
            pltpu.VMEM((64, H), jnp.float32),
            pltpu.VMEM((64,), jnp.int32),
            pltpu.VMEM((64,), jnp.int32),
            pltpu.SemaphoreType.DMA,
            pltpu.SemaphoreType.DMA,
        ],
    )(tokens, pos0, pos1)

    shared_out = pl.pallas_call(
        _shared_ffn_kernel,
        grid=(4,),
        in_specs=[
            pl.BlockSpec((T // 4, H), lambda i: (i, 0)),
            pl.BlockSpec((FF, H), lambda i: (0, 0)),
            pl.BlockSpec((FF, H), lambda i: (0, 0)),
            pl.BlockSpec((H, FF), lambda i: (0, 0)),
        ],
        out_specs=pl.BlockSpec((T // 4, H), lambda i: (i, 0)),
        out_shape=jax.ShapeDtypeStruct((T, H), jnp.float32),
    )(tokens, shared_gate, shared_up, shared_down)

    exp_out = pl.pallas_call(
        _ffn_kernel,
        grid_spec=pltpu.PrefetchScalarGridSpec(
            num_scalar_prefetch=2,
            grid=(NBLK,),
            in_specs=[
                pl.BlockSpec((BT, H), lambda bb, be_, v_: (bb, 0)),
                pl.BlockSpec((1, FF, H), lambda bb, be_, v_: (be_[bb], 0, 0)),
                pl.BlockSpec((1, FF, H), lambda bb, be_, v_: (be_[bb], 0, 0)),
                pl.BlockSpec((1, H, FF), lambda bb, be_, v_: (be_[bb], 0, 0)),
            ],
            out_specs=pl.BlockSpec((BT, H), lambda bb, be_, v_: (bb, 0)),
        ),
        out_shape=jax.ShapeDtypeStruct((NPOS, H), jnp.float32),
    )(be, valid, xs, gate_w, up_w, down_w)

    out = pl.kernel(
        _combine_kernel,
        mesh=mesh,
        out_type=jax.ShapeDtypeStruct((T, H), jnp.float32),
        scratch_types=[
            pltpu.VMEM((CH4, H), jnp.float32),
            pltpu.VMEM((CH4, H), jnp.float32),
            pltpu.VMEM((CH4, H), jnp.float32),
            pltpu.VMEM((CH4, H), jnp.float32),
            pltpu.VMEM((CH4, H), jnp.float32),
            pltpu.VMEM((CH4, H), jnp.float32),
            pltpu.VMEM((CH4, H), jnp.float32),
            pltpu.VMEM((64,), jnp.int32),
            pltpu.VMEM((64,), jnp.int32),
            pltpu.VMEM((80,), jnp.float32),
            pltpu.VMEM((80,), jnp.float32),
            pltpu.SemaphoreType.DMA,
            pltpu.SemaphoreType.DMA,
        ],
    )(exp_out, shared_out, pos0, pos1, w0, w1)

    return out.reshape(b, s, h)
